# baseline (speedup 1.0000x reference)
import functools

import jax
import jax.numpy as jnp
from jax import lax
from jax.experimental import pallas as pl
from jax.experimental.pallas import tpu as pltpu
from jax.experimental.pallas import tpu_sc as plsc

B = 16384
NE = 1000000
DV = 64
DPS = 32
NTC = 16
L = 16
SPT = B // NTC
REFS = 2 * SPT
SLABW = 208 * 128
NSLAB = 37
M2_LO = NSLAB * SLABW
M2_W = 14848
EB_LO = 999936
SMW = DPS * SLABW


def _partial_scores(e1_idx, r_idx, e2_idx, entT, relT, tailT):
    mesh = plsc.VectorSubcoreMesh(core_axis_name="c", subcore_axis_name="s")

    @functools.partial(
        pl.kernel,
        mesh=mesh,
        out_type=jax.ShapeDtypeStruct((2, B), jnp.float32),
        scratch_types=[
            pltpu.VMEM((REFS,), jnp.int32),
            pltpu.VMEM((SPT,), jnp.int32),
            pltpu.VMEM((1024,), jnp.float32),
            pltpu.VMEM((REFS,), jnp.int32),
            pltpu.VMEM((REFS,), jnp.int32),
            pltpu.VMEM((4, 128), jnp.int32),
            pltpu.VMEM((DPS * L,), jnp.float32),
            pltpu.VMEM((REFS * DPS,), jnp.float32),
            pltpu.VMEM((SPT,), jnp.float32),
            pltpu.VMEM_SHARED((SMW,), jnp.float32),
            pltpu.SemaphoreType.DMA,
            pltpu.SemaphoreType.DMA,
        ],
        compiler_params=pltpu.CompilerParams(
            needs_layout_passes=False, use_tc_tiling_on_sc=True),
    )
    def k(e1_hbm, r_hbm, e2_hbm, entT_hbm, relT_hbm, tailT_hbm, out_hbm,
          ids_v, rid_v, rrow_v, hitj_v, hits_v, eidx_v, eval_v, vals_v,
          out_v, sm, s1, s2):
        cid = lax.axis_index("c")
        tid = lax.axis_index("s")
        sbase = tid * SPT
        rbase = cid * DPS
        pltpu.sync_copy(e1_hbm.at[pl.ds(sbase, SPT)], ids_v.at[pl.ds(0, SPT)])
        pltpu.sync_copy(e2_hbm.at[pl.ds(sbase, SPT)],
                        ids_v.at[pl.ds(SPT, SPT)])
        pltpu.sync_copy(r_hbm.at[pl.ds(sbase, SPT)], rid_v)

        lane = lax.iota(jnp.int32, L)
        zero_cnt = jnp.zeros((L,), jnp.int32)

        def z_body(ch, carry):
            hitj_v[pl.ds(ch * L, L)] = zero_cnt
            hits_v[pl.ds(ch * L, L)] = zero_cnt
            return carry
        lax.fori_loop(0, REFS // L, z_body, 0)

        def load_half(tbl, src_lo, rs, dst_off, w):
            r0 = rbase + 2 * tid
            c1 = pltpu.async_copy(
                tbl.at[r0, pl.ds(src_lo, w)],
                sm.at[pl.ds((2 * tid) * rs + dst_off, w)], s1)
            c2 = pltpu.async_copy(
                tbl.at[r0 + 1, pl.ds(src_lo, w)],
                sm.at[pl.ds((2 * tid + 1) * rs + dst_off, w)], s1)
            return (c1, c2)

        def scan_refs(lo, hi, dref):
            def ch_body(ch, cnt_v):
                ids16 = ids_v[pl.ds(ch * L, L)]
                m = (ids16 >= lo) & (ids16 < hi)
                mi = jnp.where(m, 1, 0)
                pos = cnt_v + plsc.cumsum(mi) - 1
                rec = ((ids16 - lo) << 11) | (ch * L + lane)
                plsc.store_scatter(dref, [pos], rec, mask=m)
                return cnt_v + plsc.all_reduce_population_count(m)
            return lax.fori_loop(0, REFS // L, ch_body, zero_cnt)

        def extract(cnt_v, dref, rs, boff):
            nch = jnp.max(cnt_v + (L - 1)) // L

            def h_body(hc, carry):
                h16 = dref[pl.ds(hc * L, L)]
                j16 = (h16 >> 11) + boff
                s16 = h16 & 2047
                mrem = lane < (cnt_v - hc * L)
                for r in range(DPS):
                    eidx_v[r // 8, pl.ds((r % 8) * L, L)] = j16 + r * rs
                cps = [pltpu.async_copy(
                    sm.at[eidx_v.at[q]],
                    eval_v.at[pl.ds(q * 128, 128)], s2) for q in range(4)]
                for cp in cps:
                    cp.wait()
                vbase = s16 * DPS
                for r in range(DPS):
                    v16 = eval_v[pl.ds(r * L, L)]
                    plsc.store_scatter(vals_v, [vbase + r], v16, mask=mrem)
                return carry
            lax.fori_loop(0, nch, h_body, 0)

        def process(tbl, lo, width, hi):
            hw = width // 2
            cpsA = load_half(tbl, lo, width, 0, hw)
            cntA = scan_refs(lo, lo + hw, hitj_v)
            cntB = scan_refs(lo + hw, hi, hits_v)
            for cp in cpsA:
                cp.wait()
            plsc.subcore_barrier()
            cpsB = load_half(tbl, lo + hw, width, hw, width - hw)
            extract(cntA, hitj_v, width, 0)
            for cp in cpsB:
                cp.wait()
            plsc.subcore_barrier()
            extract(cntB, hits_v, width, hw)
            plsc.subcore_barrier()

        def s_body(s, carry):
            lo = s * SLABW
            process(entT_hbm, lo, SLABW, lo + SLABW)
            return carry
        lax.fori_loop(0, NSLAB, s_body, 0)
        process(entT_hbm, M2_LO, M2_W, M2_LO + M2_W)
        cps = load_half(tailT_hbm, 0, 128, 0, 128)
        cnt_v = scan_refs(EB_LO, NE, hitj_v)
        for cp in cps:
            cp.wait()
        plsc.subcore_barrier()
        extract(cnt_v, hitj_v, 128, 0)
        plsc.subcore_barrier()

        for r in range(DPS):
            pltpu.sync_copy(relT_hbm.at[rbase + r, pl.ds(0, 1024)], rrow_v)

            def g_body(g, carry, r=r):
                base16 = (g * L + lane) * DPS + r
                rid16 = rid_v[pl.ds(g * L, L)]
                v1 = plsc.load_gather(vals_v, [base16])
                v2 = plsc.load_gather(vals_v, [base16 + SPT * DPS])
                vr = plsc.load_gather(rrow_v, [rid16])
                prod = v1 * v2 * vr
                if r > 0:
                    prod = prod + out_v[pl.ds(g * L, L)]
                out_v[pl.ds(g * L, L)] = prod
                return carry
            lax.fori_loop(0, SPT // L, g_body, 0)
        pltpu.sync_copy(out_v, out_hbm.at[cid, pl.ds(sbase, SPT)])

    return k(e1_idx, r_idx, e2_idx, entT, relT, tailT)


def kernel(e1_idx, r_idx, e2_idx, entity_table, relation_table):
    e1 = e1_idx.astype(jnp.int32)
    r = r_idx.astype(jnp.int32)
    e2 = e2_idx.astype(jnp.int32)
    entT = entity_table.T
    relT = jnp.pad(relation_table, ((0, 24), (0, 0))).T
    tailT = jnp.pad(entity_table[EB_LO:].T, ((0, 0), (0, 64)))
    p = _partial_scores(e1, r, e2, entT, relT, tailT)
    out = jax.nn.sigmoid(p[0] + p[1])
    return (out, jnp.float32(0.0))


# speedup vs baseline: 1.0268x; 1.0268x over previous
import functools

import jax
import jax.numpy as jnp
from jax import lax
from jax.experimental import pallas as pl
from jax.experimental.pallas import tpu as pltpu
from jax.experimental.pallas import tpu_sc as plsc

B = 16384
NE = 1000000
DV = 64
DPS = 32
NTC = 16
L = 16
SPT = B // NTC
REFS = 2 * SPT
SLABW = 208 * 128
NSLAB = 37
M2_LO = NSLAB * SLABW
M2_W = 14848
EB_LO = 999936
SMW = DPS * SLABW


def _partial_scores(e1_idx, r_idx, e2_idx, entT, relT, tailT):
    mesh = plsc.VectorSubcoreMesh(core_axis_name="c", subcore_axis_name="s")

    @functools.partial(
        pl.kernel,
        mesh=mesh,
        out_type=jax.ShapeDtypeStruct((2, B), jnp.float32),
        scratch_types=[
            pltpu.VMEM((REFS,), jnp.int32),
            pltpu.VMEM((SPT,), jnp.int32),
            pltpu.VMEM((1024,), jnp.float32),
            pltpu.VMEM((REFS,), jnp.int32),
            pltpu.VMEM((REFS,), jnp.int32),
            pltpu.VMEM((4, 128), jnp.int32),
            pltpu.VMEM((DPS * L,), jnp.float32),
            pltpu.VMEM((REFS * DPS,), jnp.float32),
            pltpu.VMEM((SPT,), jnp.float32),
            pltpu.VMEM_SHARED((SMW,), jnp.float32),
            pltpu.SemaphoreType.DMA,
            pltpu.SemaphoreType.DMA,
        ],
        compiler_params=pltpu.CompilerParams(
            needs_layout_passes=False, use_tc_tiling_on_sc=True),
    )
    def k(e1_hbm, r_hbm, e2_hbm, entT_hbm, relT_hbm, tailT_hbm, out_hbm,
          ids_v, rid_v, rrow_v, hitj_v, hits_v, eidx_v, eval_v, vals_v,
          out_v, sm, s1, s2):
        cid = lax.axis_index("c")
        tid = lax.axis_index("s")
        sbase = tid * SPT
        rbase = cid * DPS
        pltpu.sync_copy(e1_hbm.at[pl.ds(sbase, SPT)], ids_v.at[pl.ds(0, SPT)])
        pltpu.sync_copy(e2_hbm.at[pl.ds(sbase, SPT)],
                        ids_v.at[pl.ds(SPT, SPT)])
        pltpu.sync_copy(r_hbm.at[pl.ds(sbase, SPT)], rid_v)

        lane = lax.iota(jnp.int32, L)
        zero_cnt = jnp.zeros((L,), jnp.int32)

        def z_body(ch, carry):
            hitj_v[pl.ds(ch * L, L)] = zero_cnt
            return carry
        lax.fori_loop(0, REFS // L, z_body, 0)

        def load_slab(tbl, lo, width):
            r0 = rbase + 2 * tid
            c1 = pltpu.async_copy(
                tbl.at[r0, pl.ds(lo, width)],
                sm.at[pl.ds((2 * tid) * width, width)], s1)
            c2 = pltpu.async_copy(
                tbl.at[r0 + 1, pl.ds(lo, width)],
                sm.at[pl.ds((2 * tid + 1) * width, width)], s1)
            return (c1, c2)

        def scan_refs(lo, hi):
            def ch_body(ch, cnt_v):
                ids16 = ids_v[pl.ds(ch * L, L)]
                m = (ids16 >= lo) & (ids16 < hi)
                mi = jnp.where(m, 1, 0)
                pos = cnt_v + plsc.cumsum(mi) - 1
                plsc.store_scatter(hitj_v, [pos], ids16 - lo, mask=m)
                plsc.store_scatter(hits_v, [pos], ch * L + lane, mask=m)
                return cnt_v + plsc.all_reduce_population_count(m)
            return lax.fori_loop(0, REFS // L, ch_body, zero_cnt)

        def extract(cnt_v, rs):
            nch = jnp.max(cnt_v + (L - 1)) // L

            def h_body(hc, carry):
                j16 = hitj_v[pl.ds(hc * L, L)]
                s16 = hits_v[pl.ds(hc * L, L)]
                mrem = lane < (cnt_v - hc * L)
                for r in range(DPS):
                    eidx_v[r // 8, pl.ds((r % 8) * L, L)] = j16 + r * rs
                cps = [pltpu.async_copy(
                    sm.at[eidx_v.at[q]],
                    eval_v.at[pl.ds(q * 128, 128)], s2) for q in range(4)]
                for cp in cps:
                    cp.wait()
                vbase = s16 * DPS
                for r in range(DPS):
                    v16 = eval_v[pl.ds(r * L, L)]
                    plsc.store_scatter(vals_v, [vbase + r], v16, mask=mrem)
                return carry
            lax.fori_loop(0, nch, h_body, 0)

        def process(tbl, lo, width, hi):
            cps = load_slab(tbl, lo, width)
            cnt_v = scan_refs(lo, hi)
            for cp in cps:
                cp.wait()
            plsc.subcore_barrier()
            extract(cnt_v, width)
            plsc.subcore_barrier()

        def s_body(s, carry):
            lo = s * SLABW
            process(entT_hbm, lo, SLABW, lo + SLABW)
            return carry
        lax.fori_loop(0, NSLAB, s_body, 0)
        process(entT_hbm, M2_LO, M2_W, M2_LO + M2_W)
        cps = load_slab(tailT_hbm, 0, 128)
        cnt_v = scan_refs(EB_LO, NE)
        for cp in cps:
            cp.wait()
        plsc.subcore_barrier()
        extract(cnt_v, 128)
        plsc.subcore_barrier()

        for r in range(DPS):
            pltpu.sync_copy(relT_hbm.at[rbase + r, pl.ds(0, 1024)], rrow_v)

            def g_body(g, carry, r=r):
                base16 = (g * L + lane) * DPS + r
                rid16 = rid_v[pl.ds(g * L, L)]
                v1 = plsc.load_gather(vals_v, [base16])
                v2 = plsc.load_gather(vals_v, [base16 + SPT * DPS])
                vr = plsc.load_gather(rrow_v, [rid16])
                prod = v1 * v2 * vr
                if r > 0:
                    prod = prod + out_v[pl.ds(g * L, L)]
                out_v[pl.ds(g * L, L)] = prod
                return carry
            lax.fori_loop(0, SPT // L, g_body, 0)
        pltpu.sync_copy(out_v, out_hbm.at[cid, pl.ds(sbase, SPT)])

    return k(e1_idx, r_idx, e2_idx, entT, relT, tailT)


def kernel(e1_idx, r_idx, e2_idx, entity_table, relation_table):
    e1 = e1_idx.astype(jnp.int32)
    r = r_idx.astype(jnp.int32)
    e2 = e2_idx.astype(jnp.int32)
    entT = entity_table.T
    relT = jnp.pad(relation_table, ((0, 24), (0, 0))).T
    tailT = jnp.pad(entity_table[EB_LO:].T, ((0, 0), (0, 64)))
    p = _partial_scores(e1, r, e2, entT, relT, tailT)
    out = jax.nn.sigmoid(p[0] + p[1])
    return (out, jnp.float32(0.0))
